# baseline (speedup 1.0000x reference)
import jax
import jax.numpy as jnp
from jax.experimental import pallas as pl
from jax.experimental.pallas import tpu as pltpu

_CFG = [
    (512, 128, 512, 512, True),
    (128, 128, 512, 128, False),
    (128, 32, 128, 128, True),
    (32, 32, 128, 32, False),
    (32, 4, 32, 32, True),
    (4, 4, 32, 4, False),
    (4, 1, 4, 4, True),
]

_R = 512


def _gcn_body(X_ref, Z_ref, adj_e_ref, adj_v_ref, T_ref, Tt_ref,
              dv_ref, de_ref,
              W1, b1, p1, W2, b2, p2, W3, b3, p3, W4, b4, p4,
              W5, b5, p5, W6, b6, p6, W7, b7, p7,
              out_ref, hv_scr, he_scr):
    f32 = jnp.float32
    bf16 = jnp.bfloat16
    Hv = X_ref[...]
    He = Z_ref[...]

    N = X_ref.shape[0]
    E = Tt_ref.shape[0]

    Ws = (W1, W2, W3, W4, W5, W6, W7)
    bs = (b1, b2, b3, b4, b5, b6, b7)
    ps = (p1, p2, p3, p4, p5, p6, p7)

    nlayers = len(_CFG)
    for i, (iv, ov, ie, oe, node_layer) in enumerate(_CFG):
        W = Ws[i][...]
        b = bs[i][...]
        p = ps[i][...]
        last = i + 1 == nlayers
        if node_layer:
            d = jnp.dot(He, p, preferred_element_type=f32)
            TT = T_ref[...]
            mdiag = jnp.dot(TT * TT, d, preferred_element_type=f32)
            corr = dv_ref[...] * (1.0 - mdiag)
            HW = jnp.dot(Hv, W, preferred_element_type=f32)
            hv_scr[:, 0:ov] = corr * HW + b
            HWb = HW.astype(bf16)
            Sb = (Tt_ref[...] * d).astype(bf16)

            def nbody(r, _, HWb=HWb, Sb=Sb, ov=ov):
                r0 = r * _R
                Trb = T_ref[pl.ds(r0, _R), :].astype(bf16)
                multr = jnp.dot(Trb, Sb, preferred_element_type=f32)
                Ab = (multr * adj_v_ref[pl.ds(r0, _R), :]).astype(bf16)
                hv_scr[pl.ds(r0, _R), 0:ov] = (
                    hv_scr[pl.ds(r0, _R), 0:ov]
                    + jnp.dot(Ab, HWb, preferred_element_type=f32))
                return 0

            jax.lax.fori_loop(0, N // _R, nbody, 0, unroll=False)
            if last:
                out_ref[...] = jax.nn.sigmoid(hv_scr[:, 0:1])
            else:
                Hv = jnp.maximum(hv_scr[:, 0:ov], 0.0)
                He = jnp.maximum(He, 0.0)
        else:
            d = jnp.dot(Hv, p, preferred_element_type=f32)
            TTt = Tt_ref[...]
            mdiag = jnp.dot(TTt * TTt, d, preferred_element_type=f32)
            corr = de_ref[...] * (1.0 - mdiag)
            HW = jnp.dot(He, W, preferred_element_type=f32)
            he_scr[:, 0:oe] = corr * HW + b
            HWb = HW.astype(bf16)
            Sb = (T_ref[...] * d).astype(bf16)

            def ebody(r, _, HWb=HWb, Sb=Sb, oe=oe):
                r0 = r * _R
                Ttrb = Tt_ref[pl.ds(r0, _R), :].astype(bf16)
                multr = jnp.dot(Ttrb, Sb, preferred_element_type=f32)
                Ab = (multr * adj_e_ref[pl.ds(r0, _R), :]).astype(bf16)
                he_scr[pl.ds(r0, _R), 0:oe] = (
                    he_scr[pl.ds(r0, _R), 0:oe]
                    + jnp.dot(Ab, HWb, preferred_element_type=f32))
                return 0

            jax.lax.fori_loop(0, E // _R, ebody, 0, unroll=False)
            He = jnp.maximum(he_scr[:, 0:oe], 0.0)
            Hv = jnp.maximum(Hv, 0.0)


def kernel(X, Z, adj_e, adj_v, T,
           W1, b1, p1, W2, b2, p2, W3, b3, p3, W4, b4, p4,
           W5, b5, p5, W6, b6, p6, W7, b7, p7):
    N = X.shape[0]
    E = Z.shape[0]
    bs = [b1, b2, b3, b4, b5, b6, b7]
    ps = [p1, p2, p3, p4, p5, p6, p7]
    Ws = [W1, W2, W3, W4, W5, W6, W7]
    dv = jnp.diagonal(adj_v).reshape(-1, 1)
    de = jnp.diagonal(adj_e).reshape(-1, 1)
    operands = [X, Z, adj_e, adj_v, T, T.T, dv, de]
    for W, b, p in zip(Ws, bs, ps):
        operands += [W, b.reshape(1, -1), p.T]

    return pl.pallas_call(
        _gcn_body,
        out_shape=jax.ShapeDtypeStruct((N, 1), jnp.float32),
        scratch_shapes=[
            pltpu.VMEM((N, 128), jnp.float32),
            pltpu.VMEM((E, 128), jnp.float32),
        ],
        compiler_params=pltpu.CompilerParams(
            vmem_limit_bytes=100 * 1024 * 1024,
        ),
    )(*operands)


# speedup vs baseline: 1.1557x; 1.1557x over previous
import jax
import jax.numpy as jnp
from jax.experimental import pallas as pl
from jax.experimental.pallas import tpu as pltpu

_CFG = [
    (512, 128, 512, 512, True),
    (128, 128, 512, 128, False),
    (128, 32, 128, 128, True),
    (32, 32, 128, 32, False),
    (32, 4, 32, 32, True),
    (4, 4, 32, 4, False),
    (4, 1, 4, 4, True),
]


def _diag_one(mult):
    row = jax.lax.broadcasted_iota(jnp.int32, mult.shape, 0)
    col = jax.lax.broadcasted_iota(jnp.int32, mult.shape, 1)
    return jnp.where(row == col, jnp.float32(1.0), mult)


def _gcn_body(X_ref, Z_ref, adj_e_ref, adj_v_ref, T_ref, Tt_ref,
              W1, b1, p1, W2, b2, p2, W3, b3, p3, W4, b4, p4,
              W5, b5, p5, W6, b6, p6, W7, b7, p7, out_ref):
    Hv = X_ref[...]
    He = Z_ref[...]
    T = T_ref[...]
    Tt = Tt_ref[...]
    Av = adj_v_ref[...]
    Ae = adj_e_ref[...]

    Ws = (W1, W2, W3, W4, W5, W6, W7)
    bs = (b1, b2, b3, b4, b5, b6, b7)
    ps = (p1, p2, p3, p4, p5, p6, p7)

    nlayers = len(_CFG)
    for i, (iv, ov, ie, oe, node_layer) in enumerate(_CFG):
        W = Ws[i][...]
        b = bs[i][...]
        p = ps[i][...]
        if node_layer:
            d = jnp.dot(He, p, preferred_element_type=jnp.float32)
            mult = jnp.dot(T, Tt * d, preferred_element_type=jnp.float32)
            A = _diag_one(mult) * Av
            HW = jnp.dot(Hv, W, preferred_element_type=jnp.float32)
            Hv = jnp.dot(A, HW, preferred_element_type=jnp.float32) + b
        else:
            d = jnp.dot(Hv, p, preferred_element_type=jnp.float32)
            mult = jnp.dot(Tt, T * d, preferred_element_type=jnp.float32)
            A = _diag_one(mult) * Ae
            HW = jnp.dot(He, W, preferred_element_type=jnp.float32)
            He = jnp.dot(A, HW, preferred_element_type=jnp.float32) + b
        if i + 1 < nlayers:
            Hv = jnp.maximum(Hv, 0.0)
            He = jnp.maximum(He, 0.0)

    out_ref[...] = jax.nn.sigmoid(Hv)


def kernel(X, Z, adj_e, adj_v, T,
           W1, b1, p1, W2, b2, p2, W3, b3, p3, W4, b4, p4,
           W5, b5, p5, W6, b6, p6, W7, b7, p7):
    N = X.shape[0]
    Tt = T.T
    bs = [b1, b2, b3, b4, b5, b6, b7]
    ps = [p1, p2, p3, p4, p5, p6, p7]
    Ws = [W1, W2, W3, W4, W5, W6, W7]
    operands = [X, Z, adj_e, adj_v, T, Tt]
    for W, b, p in zip(Ws, bs, ps):
        operands += [W, b.reshape(1, -1), p.T]

    return pl.pallas_call(
        _gcn_body,
        out_shape=jax.ShapeDtypeStruct((N, 1), jnp.float32),
        compiler_params=pltpu.CompilerParams(
            vmem_limit_bytes=128 * 1024 * 1024,
        ),
    )(*operands)


# speedup vs baseline: 1.2809x; 1.1083x over previous
import jax
import jax.numpy as jnp
from jax.experimental import pallas as pl
from jax.experimental.pallas import tpu as pltpu

_CFG = [
    (512, 128, 512, 512, True),
    (128, 128, 512, 128, False),
    (128, 32, 128, 128, True),
    (32, 32, 128, 32, False),
    (32, 4, 32, 32, True),
    (4, 4, 32, 4, False),
    (4, 1, 4, 4, True),
]

_NT = (((1,), (1,)), ((), ()))
_TN = (((0,), (0,)), ((), ()))


def _diag_one(mult):
    row = jax.lax.broadcasted_iota(jnp.int32, mult.shape, 0)
    col = jax.lax.broadcasted_iota(jnp.int32, mult.shape, 1)
    return jnp.where(row == col, jnp.float32(1.0), mult)


def _gcn_body(X_ref, Z_ref, adj_e_ref, adj_v_ref, T_ref,
              W1, b1, p1, W2, b2, p2, W3, b3, p3, W4, b4, p4,
              W5, b5, p5, W6, b6, p6, W7, b7, p7, out_ref):
    f32 = jnp.float32
    Hv = X_ref[...]
    He = Z_ref[...]
    T = T_ref[...]
    Av = adj_v_ref[...]
    Ae = adj_e_ref[...]

    Ws = (W1, W2, W3, W4, W5, W6, W7)
    bs = (b1, b2, b3, b4, b5, b6, b7)
    ps = (p1, p2, p3, p4, p5, p6, p7)

    nlayers = len(_CFG)
    for i, (iv, ov, ie, oe, node_layer) in enumerate(_CFG):
        W = Ws[i][...]
        b = bs[i][...]
        p = ps[i][...]
        if node_layer:
            d = jnp.dot(He, p, preferred_element_type=f32)
            mult = jax.lax.dot_general(T * d.T, T, _NT,
                                       preferred_element_type=f32)
            A = _diag_one(mult) * Av
            HW = jnp.dot(Hv, W, preferred_element_type=f32)
            Hv = jnp.dot(A, HW, preferred_element_type=f32) + b
        else:
            d = jnp.dot(Hv, p, preferred_element_type=f32)
            mult = jax.lax.dot_general(T * d, T, _TN,
                                       preferred_element_type=f32)
            A = _diag_one(mult) * Ae
            HW = jnp.dot(He, W, preferred_element_type=f32)
            He = jnp.dot(A, HW, preferred_element_type=f32) + b
        if i + 1 < nlayers:
            Hv = jnp.maximum(Hv, 0.0)
            He = jnp.maximum(He, 0.0)

    out_ref[...] = jax.nn.sigmoid(Hv)


def kernel(X, Z, adj_e, adj_v, T,
           W1, b1, p1, W2, b2, p2, W3, b3, p3, W4, b4, p4,
           W5, b5, p5, W6, b6, p6, W7, b7, p7):
    N = X.shape[0]
    bs = [b1, b2, b3, b4, b5, b6, b7]
    ps = [p1, p2, p3, p4, p5, p6, p7]
    Ws = [W1, W2, W3, W4, W5, W6, W7]
    operands = [X, Z, adj_e, adj_v, T]
    for W, b, p in zip(Ws, bs, ps):
        operands += [W, b.reshape(1, -1), p.T]

    return pl.pallas_call(
        _gcn_body,
        out_shape=jax.ShapeDtypeStruct((N, 1), jnp.float32),
        compiler_params=pltpu.CompilerParams(
            vmem_limit_bytes=128 * 1024 * 1024,
        ),
    )(*operands)


# speedup vs baseline: 1.2875x; 1.0052x over previous
import jax
import jax.numpy as jnp
from jax.experimental import pallas as pl
from jax.experimental.pallas import tpu as pltpu

_CFG = [
    (512, 128, 512, 512, True),
    (128, 128, 512, 128, False),
    (128, 32, 128, 128, True),
    (32, 32, 128, 32, False),
    (32, 4, 32, 32, True),
    (4, 4, 32, 4, False),
    (4, 1, 4, 4, True),
]

_NT = (((1,), (1,)), ((), ()))
_TN = (((0,), (0,)), ((), ()))


def _diag_one(mult):
    row = jax.lax.broadcasted_iota(jnp.int32, mult.shape, 0)
    col = jax.lax.broadcasted_iota(jnp.int32, mult.shape, 1)
    return jnp.where(row == col, jnp.float32(1.0), mult)


def _gcn_body(X_ref, Z_ref, adj_e_ref, adj_v_ref, T_ref,
              W1, b1, p1, W2, b2, p2, W3, b3, p3, W4, b4, p4,
              W5, b5, p5, W6, b6, p6, W7, b7, p7, out_ref):
    f32 = jnp.float32
    bf16 = jnp.bfloat16
    Hv = X_ref[...]
    He = Z_ref[...]
    Tb = T_ref[...].astype(bf16)
    Av = adj_v_ref[...]
    Ae = adj_e_ref[...]

    Ws = (W1, W2, W3, W4, W5, W6, W7)
    bs = (b1, b2, b3, b4, b5, b6, b7)
    ps = (p1, p2, p3, p4, p5, p6, p7)

    nlayers = len(_CFG)
    for i, (iv, ov, ie, oe, node_layer) in enumerate(_CFG):
        W = Ws[i][...]
        b = bs[i][...]
        p = ps[i][...]
        if node_layer:
            d = jnp.dot(He, p, preferred_element_type=f32)
            Sb = Tb * d.T.astype(bf16)
            mult = jax.lax.dot_general(Sb, Tb, _NT,
                                       preferred_element_type=f32)
            A = (_diag_one(mult) * Av).astype(bf16)
            HW = jnp.dot(Hv, W, preferred_element_type=f32)
            Hv = jnp.dot(A, HW.astype(bf16), preferred_element_type=f32) + b
        else:
            d = jnp.dot(Hv, p, preferred_element_type=f32)
            Sb = Tb * d.astype(bf16)
            mult = jax.lax.dot_general(Sb, Tb, _TN,
                                       preferred_element_type=f32)
            A = (_diag_one(mult) * Ae).astype(bf16)
            HW = jnp.dot(He, W, preferred_element_type=f32)
            He = jnp.dot(A, HW.astype(bf16), preferred_element_type=f32) + b
        if i + 1 < nlayers:
            Hv = jnp.maximum(Hv, 0.0)
            He = jnp.maximum(He, 0.0)

    out_ref[...] = jax.nn.sigmoid(Hv)


def kernel(X, Z, adj_e, adj_v, T,
           W1, b1, p1, W2, b2, p2, W3, b3, p3, W4, b4, p4,
           W5, b5, p5, W6, b6, p6, W7, b7, p7):
    N = X.shape[0]
    bs = [b1, b2, b3, b4, b5, b6, b7]
    ps = [p1, p2, p3, p4, p5, p6, p7]
    Ws = [W1, W2, W3, W4, W5, W6, W7]
    operands = [X, Z, adj_e, adj_v, T]
    for W, b, p in zip(Ws, bs, ps):
        operands += [W, b.reshape(1, -1), p.T]

    return pl.pallas_call(
        _gcn_body,
        out_shape=jax.ShapeDtypeStruct((N, 1), jnp.float32),
        compiler_params=pltpu.CompilerParams(
            vmem_limit_bytes=128 * 1024 * 1024,
        ),
    )(*operands)
